# Initial kernel scaffold; baseline (speedup 1.0000x reference)
import jax
import jax.numpy as jnp
from jax.experimental import pallas as pl


def kernel(P, D, M, background, width, height, means3D, shs, opacities, scales, scale_modifier, rotations, viewmatrix, projmatrix, cam_pos, tanfovx, tanfovy):
    raise NotImplementedError("write your pallas kernel here")


# speedup vs baseline: 1.3688x; 1.3688x over previous
import functools

import jax
import jax.numpy as jnp
import numpy as np
from jax.experimental import pallas as pl
from jax.experimental.pallas import tpu as pltpu

SH_C0 = 0.28209479177387814
SH_C1 = 0.4886025119029199
SH_C2 = (1.0925484305920792, -1.0925484305920792, 0.31539156525252005, -1.0925484305920792, 0.5462742152960396)
SH_C3 = (-0.5900435899266435, 2.890611442640554, -0.4570457994644658, 0.3731763325901154, -0.4570457994644658, 1.445305721320277, -0.5900435899266435)

_W = 128
_H = 128
_NPIX = 512
_CH = 256


def _eval_sh3(sh, dirs):
    x = dirs[:, 0:1]; y = dirs[:, 1:2]; z = dirs[:, 2:3]
    result = SH_C0 * sh[:, 0]
    result = result - SH_C1 * y * sh[:, 1] + SH_C1 * z * sh[:, 2] - SH_C1 * x * sh[:, 3]
    xx = x * x; yy = y * y; zz = z * z
    xy = x * y; yz = y * z; xz = x * z
    result = (result + SH_C2[0] * xy * sh[:, 4] + SH_C2[1] * yz * sh[:, 5]
              + SH_C2[2] * (2.0 * zz - xx - yy) * sh[:, 6]
              + SH_C2[3] * xz * sh[:, 7] + SH_C2[4] * (xx - yy) * sh[:, 8])
    result = (result + SH_C3[0] * y * (3.0 * xx - yy) * sh[:, 9]
              + SH_C3[1] * xy * z * sh[:, 10]
              + SH_C3[2] * y * (4.0 * zz - xx - yy) * sh[:, 11]
              + SH_C3[3] * z * (2.0 * zz - 3.0 * xx - 3.0 * yy) * sh[:, 12]
              + SH_C3[4] * x * (4.0 * zz - xx - yy) * sh[:, 13]
              + SH_C3[5] * z * (xx - yy) * sh[:, 14]
              + SH_C3[6] * x * (xx - 3.0 * yy) * sh[:, 15])
    return jnp.maximum(result + 0.5, 0.0)


def _quat_to_rot(q):
    q = q / (jnp.linalg.norm(q, axis=1, keepdims=True) + 1e-8)
    r = q[:, 0]; x = q[:, 1]; y = q[:, 2]; z = q[:, 3]
    R = jnp.stack([
        1 - 2 * (y * y + z * z), 2 * (x * y - r * z), 2 * (x * z + r * y),
        2 * (x * y + r * z), 1 - 2 * (x * x + z * z), 2 * (y * z - r * x),
        2 * (x * z - r * y), 2 * (y * z + r * x), 1 - 2 * (x * x + y * y)], axis=1)
    return R.reshape(-1, 3, 3)


def _preprocess(background, width, height, means3D, shs, opacities, scales,
                scale_modifier, rotations, viewmatrix, projmatrix, cam_pos,
                tanfovx, tanfovy):
    P = means3D.shape[0]
    ones_col = jnp.ones((P, 1), dtype=jnp.float32)
    means_hom = jnp.concatenate([means3D, ones_col], axis=1)
    vm = viewmatrix.astype(jnp.float32)
    pm = projmatrix.astype(jnp.float32)
    focal_y = height / (2.0 * tanfovy)
    focal_x = width / (2.0 * tanfovx)
    p_view = means_hom @ vm
    depths = p_view[:, 2]
    p_hom = means_hom @ pm
    p_w = 1.0 / (p_hom[:, 3:4] + 1e-7)
    p_proj = p_hom[:, :3] * p_w
    Rm = _quat_to_rot(rotations)
    s = scales * scale_modifier
    Sigma = jnp.einsum('pij,pj,pkj->pik', Rm, s * s, Rm)
    t = p_view[:, :3]
    tz = t[:, 2]
    limx = 1.3 * tanfovx
    limy = 1.3 * tanfovy
    tx = jnp.clip(t[:, 0] / tz, -limx, limx) * tz
    ty = jnp.clip(t[:, 1] / tz, -limy, limy) * tz
    zero = jnp.zeros_like(tz)
    J0 = jnp.stack([focal_x / tz, zero, -focal_x * tx / (tz * tz)], axis=1)
    J1 = jnp.stack([zero, focal_y / tz, -focal_y * ty / (tz * tz)], axis=1)
    J = jnp.stack([J0, J1], axis=1)
    Wr = vm[:3, :3].T
    Tm = jnp.einsum('pij,jk->pik', J, Wr)
    cov2D = jnp.einsum('pij,pjk,plk->pil', Tm, Sigma, Tm)
    a = cov2D[:, 0, 0] + 0.3
    b = cov2D[:, 0, 1]
    c = cov2D[:, 1, 1] + 0.3
    det = a * c - b * b
    det_safe = jnp.where(jnp.abs(det) < 1e-12, 1.0, det)
    inv_det = 1.0 / det_safe
    conic = jnp.stack([c * inv_det, -b * inv_det, a * inv_det], axis=1)
    mid = 0.5 * (a + c)
    disc = jnp.sqrt(jnp.maximum(0.1, mid * mid - det))
    lam1 = mid + disc
    radius = jnp.ceil(3.0 * jnp.sqrt(jnp.maximum(lam1, 1e-8)))
    px = ((p_proj[:, 0] + 1.0) * width - 1.0) * 0.5
    py = ((p_proj[:, 1] + 1.0) * height - 1.0) * 0.5
    dirs = means3D - cam_pos[None, :]
    dirs = dirs / (jnp.linalg.norm(dirs, axis=1, keepdims=True) + 1e-8)
    rgb = _eval_sh3(shs, dirs)
    visible = (depths > 0.2) & (det > 0.0) & (radius > 0.0)
    radii = jnp.where(visible, radius, 0.0)
    return px, py, conic, rgb, opacities[:, 0], depths, visible, radii


def _composite_body(attrs_ref, rgbd_ref, out_ref):
    i = pl.program_id(0)
    npix = _NPIX
    ch = _CH
    nchunks = attrs_ref.shape[1] // ch
    pix = i * npix + jax.lax.broadcasted_iota(jnp.int32, (npix, 1), 0)
    xf = (pix // _H).astype(jnp.float32)
    yf = (pix % _H).astype(jnp.float32)
    ri = jax.lax.broadcasted_iota(jnp.int32, (ch, ch), 0)
    ci = jax.lax.broadcasted_iota(jnp.int32, (ch, ch), 1)
    cmat = (ri < ci).astype(jnp.float32)

    def body(k, carry):
        tlog, acc = carry
        a = attrs_ref[:, pl.ds(k * ch, ch)]
        px_c = a[0:1, :]; py_c = a[1:2, :]
        c0 = a[2:3, :]; c1 = a[3:4, :]; c2 = a[4:5, :]
        opc = a[5:6, :]
        dx = px_c - xf
        dy = py_c - yf
        power = (-0.5 * (c0 * dx * dx + c2 * dy * dy)) - c1 * dx * dy
        alpha = opc * jnp.exp(jnp.minimum(power, 0.0))
        alpha = jnp.minimum(alpha, 0.99)
        alpha = jnp.where((power > 0.0) | (alpha < 1.0 / 255.0), 0.0, alpha)
        el = jnp.log1p(-alpha)
        s = jax.lax.dot(el, cmat, precision=jax.lax.Precision.HIGHEST)
        tprev = jnp.exp(tlog + s)
        w = jnp.where(tprev < 1e-4, 0.0, alpha * tprev)
        acc = acc + jax.lax.dot(w, rgbd_ref[pl.ds(k * ch, ch), :],
                                precision=jax.lax.Precision.HIGHEST)
        tlog = tlog + jnp.sum(el, axis=1, keepdims=True)
        return tlog, acc

    tlog0 = jnp.zeros((npix, 1), jnp.float32)
    acc0 = jnp.zeros((npix, 8), jnp.float32)
    _, acc = jax.lax.fori_loop(0, nchunks, body, (tlog0, acc0))
    out_ref[...] = acc


def _composite(attrs, rgbd):
    npix_total = _W * _H
    grid = (npix_total // _NPIX,)
    return pl.pallas_call(
        _composite_body,
        grid=grid,
        in_specs=[
            pl.BlockSpec(attrs.shape, lambda i: (0, 0)),
            pl.BlockSpec(rgbd.shape, lambda i: (0, 0)),
        ],
        out_specs=pl.BlockSpec((_NPIX, 8), lambda i: (i, 0)),
        out_shape=jax.ShapeDtypeStruct((npix_total, 8), jnp.float32),
    )(attrs, rgbd)


def kernel(P, D, M, background, width, height, means3D, shs, opacities, scales,
           scale_modifier, rotations, viewmatrix, projmatrix, cam_pos,
           tanfovx, tanfovy):
    px, py, conic, rgb, op, depths, visible, radii = _preprocess(
        background, width, height, means3D, shs, opacities, scales,
        scale_modifier, rotations, viewmatrix, projmatrix, cam_pos,
        tanfovx, tanfovy)
    order = jnp.argsort(depths)
    op_eff = jnp.where(visible, op, 0.0)
    zeros = jnp.zeros_like(px)
    attrs = jnp.stack([px, py, conic[:, 0], conic[:, 1], conic[:, 2],
                       op_eff, zeros, zeros], axis=0)[:, order]
    ones = jnp.ones_like(px)
    rgbd = jnp.stack([rgb[:, 0], rgb[:, 1], rgb[:, 2], depths, ones,
                      zeros, zeros, zeros], axis=1)[order]
    acc = _composite(attrs, rgbd)
    accw = acc[:, 4:5]
    out_color = (acc[:, 0:3] + (1.0 - accw) * background[None, :]).reshape(_W, _H, 3)
    out_depth = acc[:, 3:4].reshape(_W, _H, 1)
    return out_color, out_depth, radii, visible


# speedup vs baseline: 1.6131x; 1.1785x over previous
import functools

import jax
import jax.numpy as jnp
import numpy as np
from jax.experimental import pallas as pl
from jax.experimental.pallas import tpu as pltpu

SH_C0 = 0.28209479177387814
SH_C1 = 0.4886025119029199
SH_C2 = (1.0925484305920792, -1.0925484305920792, 0.31539156525252005, -1.0925484305920792, 0.5462742152960396)
SH_C3 = (-0.5900435899266435, 2.890611442640554, -0.4570457994644658, 0.3731763325901154, -0.4570457994644658, 1.445305721320277, -0.5900435899266435)

_W = 128
_H = 128
_NPIX = 512
_CH = 256


def _eval_sh3(sh, dirs):
    x = dirs[:, 0:1]; y = dirs[:, 1:2]; z = dirs[:, 2:3]
    result = SH_C0 * sh[:, 0]
    result = result - SH_C1 * y * sh[:, 1] + SH_C1 * z * sh[:, 2] - SH_C1 * x * sh[:, 3]
    xx = x * x; yy = y * y; zz = z * z
    xy = x * y; yz = y * z; xz = x * z
    result = (result + SH_C2[0] * xy * sh[:, 4] + SH_C2[1] * yz * sh[:, 5]
              + SH_C2[2] * (2.0 * zz - xx - yy) * sh[:, 6]
              + SH_C2[3] * xz * sh[:, 7] + SH_C2[4] * (xx - yy) * sh[:, 8])
    result = (result + SH_C3[0] * y * (3.0 * xx - yy) * sh[:, 9]
              + SH_C3[1] * xy * z * sh[:, 10]
              + SH_C3[2] * y * (4.0 * zz - xx - yy) * sh[:, 11]
              + SH_C3[3] * z * (2.0 * zz - 3.0 * xx - 3.0 * yy) * sh[:, 12]
              + SH_C3[4] * x * (4.0 * zz - xx - yy) * sh[:, 13]
              + SH_C3[5] * z * (xx - yy) * sh[:, 14]
              + SH_C3[6] * x * (xx - 3.0 * yy) * sh[:, 15])
    return jnp.maximum(result + 0.5, 0.0)


def _quat_to_rot(q):
    q = q / (jnp.linalg.norm(q, axis=1, keepdims=True) + 1e-8)
    r = q[:, 0]; x = q[:, 1]; y = q[:, 2]; z = q[:, 3]
    R = jnp.stack([
        1 - 2 * (y * y + z * z), 2 * (x * y - r * z), 2 * (x * z + r * y),
        2 * (x * y + r * z), 1 - 2 * (x * x + z * z), 2 * (y * z - r * x),
        2 * (x * z - r * y), 2 * (y * z + r * x), 1 - 2 * (x * x + y * y)], axis=1)
    return R.reshape(-1, 3, 3)


def _preprocess(background, width, height, means3D, shs, opacities, scales,
                scale_modifier, rotations, viewmatrix, projmatrix, cam_pos,
                tanfovx, tanfovy):
    P = means3D.shape[0]
    ones_col = jnp.ones((P, 1), dtype=jnp.float32)
    means_hom = jnp.concatenate([means3D, ones_col], axis=1)
    vm = viewmatrix.astype(jnp.float32)
    pm = projmatrix.astype(jnp.float32)
    focal_y = height / (2.0 * tanfovy)
    focal_x = width / (2.0 * tanfovx)
    p_view = means_hom @ vm
    depths = p_view[:, 2]
    p_hom = means_hom @ pm
    p_w = 1.0 / (p_hom[:, 3:4] + 1e-7)
    p_proj = p_hom[:, :3] * p_w
    Rm = _quat_to_rot(rotations)
    s = scales * scale_modifier
    Sigma = jnp.einsum('pij,pj,pkj->pik', Rm, s * s, Rm)
    t = p_view[:, :3]
    tz = t[:, 2]
    limx = 1.3 * tanfovx
    limy = 1.3 * tanfovy
    tx = jnp.clip(t[:, 0] / tz, -limx, limx) * tz
    ty = jnp.clip(t[:, 1] / tz, -limy, limy) * tz
    zero = jnp.zeros_like(tz)
    J0 = jnp.stack([focal_x / tz, zero, -focal_x * tx / (tz * tz)], axis=1)
    J1 = jnp.stack([zero, focal_y / tz, -focal_y * ty / (tz * tz)], axis=1)
    J = jnp.stack([J0, J1], axis=1)
    Wr = vm[:3, :3].T
    Tm = jnp.einsum('pij,jk->pik', J, Wr)
    cov2D = jnp.einsum('pij,pjk,plk->pil', Tm, Sigma, Tm)
    a = cov2D[:, 0, 0] + 0.3
    b = cov2D[:, 0, 1]
    c = cov2D[:, 1, 1] + 0.3
    det = a * c - b * b
    det_safe = jnp.where(jnp.abs(det) < 1e-12, 1.0, det)
    inv_det = 1.0 / det_safe
    conic = jnp.stack([c * inv_det, -b * inv_det, a * inv_det], axis=1)
    mid = 0.5 * (a + c)
    disc = jnp.sqrt(jnp.maximum(0.1, mid * mid - det))
    lam1 = mid + disc
    radius = jnp.ceil(3.0 * jnp.sqrt(jnp.maximum(lam1, 1e-8)))
    px = ((p_proj[:, 0] + 1.0) * width - 1.0) * 0.5
    py = ((p_proj[:, 1] + 1.0) * height - 1.0) * 0.5
    dirs = means3D - cam_pos[None, :]
    dirs = dirs / (jnp.linalg.norm(dirs, axis=1, keepdims=True) + 1e-8)
    rgb = _eval_sh3(shs, dirs)
    visible = (depths > 0.2) & (det > 0.0) & (radius > 0.0)
    radii = jnp.where(visible, radius, 0.0)
    return px, py, conic, rgb, opacities[:, 0], depths, visible, radii


def _composite_body(attrs_ref, rgbd_ref, out_ref):
    i = pl.program_id(0)
    npix = _NPIX
    ch = _CH
    nchunks = attrs_ref.shape[1] // ch
    pix = i * npix + jax.lax.broadcasted_iota(jnp.int32, (npix, 1), 0)
    xf = (pix // _H).astype(jnp.float32)
    yf = (pix % _H).astype(jnp.float32)

    def shift_fill1(t, sh):
        return jnp.concatenate(
            [jnp.full((t.shape[0], sh), 1.0, t.dtype), t[:, :t.shape[1] - sh]],
            axis=1)

    def body(k, carry):
        tcar, acc = carry
        a = attrs_ref[:, pl.ds(k * ch, ch)]
        px_c = a[0:1, :]; py_c = a[1:2, :]
        c0 = a[2:3, :]; c1 = a[3:4, :]; c2 = a[4:5, :]
        opc = a[5:6, :]
        dx = px_c - xf
        dy = py_c - yf
        power = (-0.5 * (c0 * dx * dx + c2 * dy * dy)) - c1 * dx * dy
        alpha = opc * jnp.exp(jnp.minimum(power, 0.0))
        alpha = jnp.minimum(alpha, 0.99)
        alpha = jnp.where((power > 0.0) | (alpha < 1.0 / 255.0), 0.0, alpha)
        t = 1.0 - alpha
        sh = 1
        while sh < ch:
            t = t * shift_fill1(t, sh)
            sh *= 2
        tprev = tcar * shift_fill1(t, 1)
        w = jnp.where(tprev < 1e-4, 0.0, alpha * tprev)
        acc = acc + jax.lax.dot(w, rgbd_ref[pl.ds(k * ch, ch), :],
                                precision=jax.lax.Precision.HIGHEST)
        tcar = tcar * t[:, ch - 1:ch]
        return tcar, acc

    tcar0 = jnp.ones((npix, 1), jnp.float32)
    acc0 = jnp.zeros((npix, 8), jnp.float32)
    _, acc = jax.lax.fori_loop(0, nchunks, body, (tcar0, acc0))
    out_ref[...] = acc


def _composite(attrs, rgbd):
    npix_total = _W * _H
    grid = (npix_total // _NPIX,)
    return pl.pallas_call(
        _composite_body,
        grid=grid,
        in_specs=[
            pl.BlockSpec(attrs.shape, lambda i: (0, 0)),
            pl.BlockSpec(rgbd.shape, lambda i: (0, 0)),
        ],
        out_specs=pl.BlockSpec((_NPIX, 8), lambda i: (i, 0)),
        out_shape=jax.ShapeDtypeStruct((npix_total, 8), jnp.float32),
    )(attrs, rgbd)


def kernel(P, D, M, background, width, height, means3D, shs, opacities, scales,
           scale_modifier, rotations, viewmatrix, projmatrix, cam_pos,
           tanfovx, tanfovy):
    px, py, conic, rgb, op, depths, visible, radii = _preprocess(
        background, width, height, means3D, shs, opacities, scales,
        scale_modifier, rotations, viewmatrix, projmatrix, cam_pos,
        tanfovx, tanfovy)
    order = jnp.argsort(depths)
    op_eff = jnp.where(visible, op, 0.0)
    zeros = jnp.zeros_like(px)
    attrs = jnp.stack([px, py, conic[:, 0], conic[:, 1], conic[:, 2],
                       op_eff, zeros, zeros], axis=0)[:, order]
    ones = jnp.ones_like(px)
    rgbd = jnp.stack([rgb[:, 0], rgb[:, 1], rgb[:, 2], depths, ones,
                      zeros, zeros, zeros], axis=1)[order]
    acc = _composite(attrs, rgbd)
    accw = acc[:, 4:5]
    out_color = (acc[:, 0:3] + (1.0 - accw) * background[None, :]).reshape(_W, _H, 3)
    out_depth = acc[:, 3:4].reshape(_W, _H, 1)
    return out_color, out_depth, radii, visible


# speedup vs baseline: 4.1603x; 2.5790x over previous
import functools

import jax
import jax.numpy as jnp
import numpy as np
from jax.experimental import pallas as pl
from jax.experimental.pallas import tpu as pltpu

SH_C0 = 0.28209479177387814
SH_C1 = 0.4886025119029199
SH_C2 = (1.0925484305920792, -1.0925484305920792, 0.31539156525252005, -1.0925484305920792, 0.5462742152960396)
SH_C3 = (-0.5900435899266435, 2.890611442640554, -0.4570457994644658, 0.3731763325901154, -0.4570457994644658, 1.445305721320277, -0.5900435899266435)

_W = 128
_H = 128
_NPIX = 512
_CH = 256


def _eval_sh3(sh, dirs):
    x = dirs[:, 0:1]; y = dirs[:, 1:2]; z = dirs[:, 2:3]
    result = SH_C0 * sh[:, 0]
    result = result - SH_C1 * y * sh[:, 1] + SH_C1 * z * sh[:, 2] - SH_C1 * x * sh[:, 3]
    xx = x * x; yy = y * y; zz = z * z
    xy = x * y; yz = y * z; xz = x * z
    result = (result + SH_C2[0] * xy * sh[:, 4] + SH_C2[1] * yz * sh[:, 5]
              + SH_C2[2] * (2.0 * zz - xx - yy) * sh[:, 6]
              + SH_C2[3] * xz * sh[:, 7] + SH_C2[4] * (xx - yy) * sh[:, 8])
    result = (result + SH_C3[0] * y * (3.0 * xx - yy) * sh[:, 9]
              + SH_C3[1] * xy * z * sh[:, 10]
              + SH_C3[2] * y * (4.0 * zz - xx - yy) * sh[:, 11]
              + SH_C3[3] * z * (2.0 * zz - 3.0 * xx - 3.0 * yy) * sh[:, 12]
              + SH_C3[4] * x * (4.0 * zz - xx - yy) * sh[:, 13]
              + SH_C3[5] * z * (xx - yy) * sh[:, 14]
              + SH_C3[6] * x * (xx - 3.0 * yy) * sh[:, 15])
    return jnp.maximum(result + 0.5, 0.0)


def _quat_to_rot(q):
    q = q / (jnp.linalg.norm(q, axis=1, keepdims=True) + 1e-8)
    r = q[:, 0]; x = q[:, 1]; y = q[:, 2]; z = q[:, 3]
    R = jnp.stack([
        1 - 2 * (y * y + z * z), 2 * (x * y - r * z), 2 * (x * z + r * y),
        2 * (x * y + r * z), 1 - 2 * (x * x + z * z), 2 * (y * z - r * x),
        2 * (x * z - r * y), 2 * (y * z + r * x), 1 - 2 * (x * x + y * y)], axis=1)
    return R.reshape(-1, 3, 3)


def _preprocess(background, width, height, means3D, shs, opacities, scales,
                scale_modifier, rotations, viewmatrix, projmatrix, cam_pos,
                tanfovx, tanfovy):
    P = means3D.shape[0]
    ones_col = jnp.ones((P, 1), dtype=jnp.float32)
    means_hom = jnp.concatenate([means3D, ones_col], axis=1)
    vm = viewmatrix.astype(jnp.float32)
    pm = projmatrix.astype(jnp.float32)
    focal_y = height / (2.0 * tanfovy)
    focal_x = width / (2.0 * tanfovx)
    p_view = means_hom @ vm
    depths = p_view[:, 2]
    p_hom = means_hom @ pm
    p_w = 1.0 / (p_hom[:, 3:4] + 1e-7)
    p_proj = p_hom[:, :3] * p_w
    Rm = _quat_to_rot(rotations)
    s = scales * scale_modifier
    Sigma = jnp.einsum('pij,pj,pkj->pik', Rm, s * s, Rm)
    t = p_view[:, :3]
    tz = t[:, 2]
    limx = 1.3 * tanfovx
    limy = 1.3 * tanfovy
    tx = jnp.clip(t[:, 0] / tz, -limx, limx) * tz
    ty = jnp.clip(t[:, 1] / tz, -limy, limy) * tz
    zero = jnp.zeros_like(tz)
    J0 = jnp.stack([focal_x / tz, zero, -focal_x * tx / (tz * tz)], axis=1)
    J1 = jnp.stack([zero, focal_y / tz, -focal_y * ty / (tz * tz)], axis=1)
    J = jnp.stack([J0, J1], axis=1)
    Wr = vm[:3, :3].T
    Tm = jnp.einsum('pij,jk->pik', J, Wr)
    cov2D = jnp.einsum('pij,pjk,plk->pil', Tm, Sigma, Tm)
    a = cov2D[:, 0, 0] + 0.3
    b = cov2D[:, 0, 1]
    c = cov2D[:, 1, 1] + 0.3
    det = a * c - b * b
    det_safe = jnp.where(jnp.abs(det) < 1e-12, 1.0, det)
    inv_det = 1.0 / det_safe
    conic = jnp.stack([c * inv_det, -b * inv_det, a * inv_det], axis=1)
    mid = 0.5 * (a + c)
    disc = jnp.sqrt(jnp.maximum(0.1, mid * mid - det))
    lam1 = mid + disc
    radius = jnp.ceil(3.0 * jnp.sqrt(jnp.maximum(lam1, 1e-8)))
    px = ((p_proj[:, 0] + 1.0) * width - 1.0) * 0.5
    py = ((p_proj[:, 1] + 1.0) * height - 1.0) * 0.5
    dirs = means3D - cam_pos[None, :]
    dirs = dirs / (jnp.linalg.norm(dirs, axis=1, keepdims=True) + 1e-8)
    rgb = _eval_sh3(shs, dirs)
    visible = (depths > 0.2) & (det > 0.0) & (radius > 0.0)
    radii = jnp.where(visible, radius, 0.0)
    return px, py, conic, rgb, opacities[:, 0], depths, visible, radii


def _composite_body(attrs_ref, rgbd_ref, out_ref):
    i = pl.program_id(0)
    npix = _NPIX
    ch = _CH
    nchunks = attrs_ref.shape[1] // ch
    pix = i * npix + jax.lax.broadcasted_iota(jnp.int32, (npix, 1), 0)
    xf = (pix // _H).astype(jnp.float32)
    yf = (pix % _H).astype(jnp.float32)

    def shift_fill1(t, sh):
        return jnp.concatenate(
            [jnp.full((t.shape[0], sh), 1.0, t.dtype), t[:, :t.shape[1] - sh]],
            axis=1)

    def body(k, carry):
        tcar, acc = carry
        a = attrs_ref[:, pl.ds(k * ch, ch)]
        px_c = a[0:1, :]; py_c = a[1:2, :]
        c0 = a[2:3, :]; c1 = a[3:4, :]; c2 = a[4:5, :]
        opc = a[5:6, :]
        dx = px_c - xf
        dy = py_c - yf
        power = (-0.5 * (c0 * dx * dx + c2 * dy * dy)) - c1 * dx * dy
        alpha = opc * jnp.exp(jnp.minimum(power, 0.0))
        alpha = jnp.minimum(alpha, 0.99)
        alpha = jnp.where((power > 0.0) | (alpha < 1.0 / 255.0), 0.0, alpha)
        t = 1.0 - alpha
        sh = 1
        while sh < ch:
            t = t * shift_fill1(t, sh)
            sh *= 2
        tprev = tcar * shift_fill1(t, 1)
        w = jnp.where(tprev < 1e-4, 0.0, alpha * tprev)
        acc = acc + jax.lax.dot(w, rgbd_ref[pl.ds(k * ch, ch), :],
                                precision=jax.lax.Precision.HIGHEST)
        tcar = tcar * t[:, ch - 1:ch]
        return tcar, acc

    tcar0 = jnp.ones((npix, 1), jnp.float32)
    acc0 = jnp.zeros((npix, 8), jnp.float32)
    _, acc = jax.lax.fori_loop(0, nchunks, body, (tcar0, acc0))
    out_ref[...] = acc


def _composite(attrs, rgbd):
    npix_total = _W * _H
    grid = (npix_total // _NPIX,)
    return pl.pallas_call(
        _composite_body,
        grid=grid,
        in_specs=[
            pl.BlockSpec(attrs.shape, lambda i: (0, 0)),
            pl.BlockSpec(rgbd.shape, lambda i: (0, 0)),
        ],
        out_specs=pl.BlockSpec((_NPIX, 8), lambda i: (i, 0)),
        out_shape=jax.ShapeDtypeStruct((npix_total, 8), jnp.float32),
    )(attrs, rgbd)


_TILE = 16
_TGRID = _W // _TILE
_NTILES = _TGRID * _TGRID
_KSLOT = 9
_TPIX = _TILE * _TILE


def _tile_composite_body(starts_ref, binned_ref, out_ref):
    t = pl.program_id(0)
    start = starts_ref[t]
    end = starts_ref[t + 1]
    rr = jax.lax.broadcasted_iota(jnp.int32, (_TPIX, 1), 0)
    xf = ((t % _TGRID) * _TILE + rr // _TILE).astype(jnp.float32)
    yf = ((t // _TGRID) * _TILE + rr % _TILE).astype(jnp.float32)
    lane = jax.lax.broadcasted_iota(jnp.int32, (1, _CH), 1)

    def shift_fill1(v, sh):
        return jnp.concatenate(
            [jnp.full((v.shape[0], sh), 1.0, v.dtype), v[:, :v.shape[1] - sh]],
            axis=1)

    def chunk(j, carry):
        tcar, acc = carry
        a = binned_ref[j]
        o = j * _CH + lane
        valid = (o >= start) & (o < end)
        px_c = a[0:1, :]; py_c = a[1:2, :]
        c0 = a[2:3, :]; c1 = a[3:4, :]; c2 = a[4:5, :]
        opc = a[5:6, :]
        dx = px_c - xf
        dy = py_c - yf
        power = (-0.5 * (c0 * dx * dx + c2 * dy * dy)) - c1 * dx * dy
        alpha = opc * jnp.exp(jnp.minimum(power, 0.0))
        alpha = jnp.minimum(alpha, 0.99)
        alpha = jnp.where((power > 0.0) | (alpha < 1.0 / 255.0) | (~valid),
                          0.0, alpha)
        tv = 1.0 - alpha
        sh = 1
        while sh < _CH:
            tv = tv * shift_fill1(tv, sh)
            sh *= 2
        tprev = tcar * shift_fill1(tv, 1)
        w = jnp.where(tprev < 1e-4, 0.0, alpha * tprev)
        acc = acc + jax.lax.dot_general(
            w, a[6:14, :], (((1,), (1,)), ((), ())),
            precision=jax.lax.Precision.HIGHEST)
        tcar = tcar * tv[:, _CH - 1:_CH]
        return tcar, acc

    j0 = start // _CH
    j1 = (end + _CH - 1) // _CH
    tcar0 = jnp.ones((_TPIX, 1), jnp.float32)
    acc0 = jnp.zeros((_TPIX, 8), jnp.float32)
    _, acc = jax.lax.fori_loop(j0, j1, chunk, (tcar0, acc0))
    out_ref[0] = acc


def _tile_composite(starts, binned):
    grid_spec = pltpu.PrefetchScalarGridSpec(
        num_scalar_prefetch=1,
        grid=(_NTILES,),
        in_specs=[pl.BlockSpec(binned.shape, lambda t, s: (0, 0, 0))],
        out_specs=pl.BlockSpec((1, _TPIX, 8), lambda t, s: (t, 0, 0)),
    )
    return pl.pallas_call(
        _tile_composite_body,
        grid_spec=grid_spec,
        out_shape=jax.ShapeDtypeStruct((_NTILES, _TPIX, 8), jnp.float32),
    )(starts, binned)


def kernel(P, D, M, background, width, height, means3D, shs, opacities, scales,
           scale_modifier, rotations, viewmatrix, projmatrix, cam_pos,
           tanfovx, tanfovy):
    px, py, conic, rgb, op, depths, visible, radii = _preprocess(
        background, width, height, means3D, shs, opacities, scales,
        scale_modifier, rotations, viewmatrix, projmatrix, cam_pos,
        tanfovx, tanfovy)
    order = jnp.argsort(depths)
    op_eff = jnp.where(visible, op, 0.0)
    zeros = jnp.zeros_like(px)
    ones = jnp.ones_like(px)
    attrs16 = jnp.stack(
        [px, py, conic[:, 0], conic[:, 1], conic[:, 2], op_eff,
         rgb[:, 0], rgb[:, 1], rgb[:, 2], depths, ones,
         zeros, zeros, zeros, zeros, zeros], axis=0)[:, order]

    cA = attrs16[2]; cB = attrs16[3]; cC = attrs16[4]; ops = attrs16[5]
    midc = 0.5 * (cA + cC)
    detc = cA * cC - cB * cB
    lminc = midc - jnp.sqrt(jnp.maximum(midc * midc - detc, 0.0))
    thresh = 2.0 * jnp.log(255.0 * jnp.maximum(ops, 1e-30))
    rad = jnp.sqrt(jnp.maximum(thresh, 0.0)
                   / jnp.maximum(lminc, 1e-30)) * 1.001 + 0.1
    ok = (ops > 0.0) & (thresh > 0.0) & (lminc > 0.0)
    safe = jnp.all(jnp.where(ok, rad <= jnp.float32(_TILE), True))

    pxs = attrs16[0]; pys = attrs16[1]
    inv_t = 1.0 / _TILE
    tx0 = jnp.clip(jnp.floor((pxs - rad) * inv_t), 0, _TGRID - 1).astype(jnp.int32)
    tx1 = jnp.clip(jnp.floor((pxs + rad) * inv_t), 0, _TGRID - 1).astype(jnp.int32)
    ty0 = jnp.clip(jnp.floor((pys - rad) * inv_t), 0, _TGRID - 1).astype(jnp.int32)
    ty1 = jnp.clip(jnp.floor((pys + rad) * inv_t), 0, _TGRID - 1).astype(jnp.int32)
    Pn = pxs.shape[0]
    rank = jnp.arange(Pn, dtype=jnp.int32)
    sent = jnp.int32(_NTILES << 13)
    keys = []
    for s in range(_KSLOT):
        tx = tx0 + s % 3
        ty = ty0 + s // 3
        v = ok & (tx <= tx1) & (ty <= ty1)
        keys.append(jnp.where(v, ((ty * _TGRID + tx) << 13) | rank, sent))
    keys = jnp.sort(jnp.stack(keys).ravel())
    tile_arr = keys >> 13
    starts = jnp.searchsorted(
        tile_arr, jnp.arange(_NTILES + 1, dtype=jnp.int32)).astype(jnp.int32)
    idx = keys & (Pn - 1)
    ncap = (_KSLOT * Pn) // _CH
    binned = attrs16[:, idx].reshape(16, ncap, _CH).swapaxes(0, 1)

    def tiled_path():
        acc = _tile_composite(starts, binned)
        a = acc.reshape(_TGRID, _TGRID, _TILE, _TILE, 8)
        return a.transpose(1, 2, 0, 3, 4).reshape(_W * _H, 8)

    def dense_path():
        rgbd = attrs16[6:11].T
        rgbd = jnp.concatenate([rgbd, jnp.zeros((Pn, 3), jnp.float32)], axis=1)
        return _composite(attrs16[0:8], rgbd)

    acc = jax.lax.cond(safe, tiled_path, dense_path)
    accw = acc[:, 4:5]
    out_color = (acc[:, 0:3] + (1.0 - accw) * background[None, :]).reshape(_W, _H, 3)
    out_depth = acc[:, 3:4].reshape(_W, _H, 1)
    return out_color, out_depth, radii, visible


# speedup vs baseline: 5.8057x; 1.3955x over previous
import functools

import jax
import jax.numpy as jnp
import numpy as np
from jax.experimental import pallas as pl
from jax.experimental.pallas import tpu as pltpu

SH_C0 = 0.28209479177387814
SH_C1 = 0.4886025119029199
SH_C2 = (1.0925484305920792, -1.0925484305920792, 0.31539156525252005, -1.0925484305920792, 0.5462742152960396)
SH_C3 = (-0.5900435899266435, 2.890611442640554, -0.4570457994644658, 0.3731763325901154, -0.4570457994644658, 1.445305721320277, -0.5900435899266435)

_W = 128
_H = 128
_NPIX = 512
_CH = 256


def _eval_sh3(sh, dirs):
    x = dirs[:, 0:1]; y = dirs[:, 1:2]; z = dirs[:, 2:3]
    result = SH_C0 * sh[:, 0]
    result = result - SH_C1 * y * sh[:, 1] + SH_C1 * z * sh[:, 2] - SH_C1 * x * sh[:, 3]
    xx = x * x; yy = y * y; zz = z * z
    xy = x * y; yz = y * z; xz = x * z
    result = (result + SH_C2[0] * xy * sh[:, 4] + SH_C2[1] * yz * sh[:, 5]
              + SH_C2[2] * (2.0 * zz - xx - yy) * sh[:, 6]
              + SH_C2[3] * xz * sh[:, 7] + SH_C2[4] * (xx - yy) * sh[:, 8])
    result = (result + SH_C3[0] * y * (3.0 * xx - yy) * sh[:, 9]
              + SH_C3[1] * xy * z * sh[:, 10]
              + SH_C3[2] * y * (4.0 * zz - xx - yy) * sh[:, 11]
              + SH_C3[3] * z * (2.0 * zz - 3.0 * xx - 3.0 * yy) * sh[:, 12]
              + SH_C3[4] * x * (4.0 * zz - xx - yy) * sh[:, 13]
              + SH_C3[5] * z * (xx - yy) * sh[:, 14]
              + SH_C3[6] * x * (xx - 3.0 * yy) * sh[:, 15])
    return jnp.maximum(result + 0.5, 0.0)


def _quat_to_rot(q):
    q = q / (jnp.linalg.norm(q, axis=1, keepdims=True) + 1e-8)
    r = q[:, 0]; x = q[:, 1]; y = q[:, 2]; z = q[:, 3]
    R = jnp.stack([
        1 - 2 * (y * y + z * z), 2 * (x * y - r * z), 2 * (x * z + r * y),
        2 * (x * y + r * z), 1 - 2 * (x * x + z * z), 2 * (y * z - r * x),
        2 * (x * z - r * y), 2 * (y * z + r * x), 1 - 2 * (x * x + y * y)], axis=1)
    return R.reshape(-1, 3, 3)


def _prep_body(g10, shsr, opar, vmr, pmr, parr, attrs, aux):
    mx = g10[0]; my = g10[1]; mz = g10[2]
    sx = g10[3]; sy = g10[4]; sz = g10[5]
    qr = g10[6]; qx = g10[7]; qy = g10[8]; qz = g10[9]
    opa = opar[0]
    v = [[vmr[i, j] for j in range(4)] for i in range(4)]
    pm = [[pmr[i, j] for j in range(4)] for i in range(4)]
    tanfovx = parr[0]; tanfovy = parr[1]; smod = parr[2]
    width = parr[3]; height = parr[4]
    cx = parr[5]; cy = parr[6]; cz = parr[7]
    focal_y = height / (2.0 * tanfovy)
    focal_x = width / (2.0 * tanfovx)
    pv0 = mx * v[0][0] + my * v[1][0] + mz * v[2][0] + v[3][0]
    pv1 = mx * v[0][1] + my * v[1][1] + mz * v[2][1] + v[3][1]
    pv2 = mx * v[0][2] + my * v[1][2] + mz * v[2][2] + v[3][2]
    depth = pv2
    ph0 = mx * pm[0][0] + my * pm[1][0] + mz * pm[2][0] + pm[3][0]
    ph1 = mx * pm[0][1] + my * pm[1][1] + mz * pm[2][1] + pm[3][1]
    ph3 = mx * pm[0][3] + my * pm[1][3] + mz * pm[2][3] + pm[3][3]
    p_w = 1.0 / (ph3 + 1e-7)
    prx = ph0 * p_w
    pry = ph1 * p_w
    qn = jnp.sqrt(qr * qr + qx * qx + qy * qy + qz * qz) + 1e-8
    qi = 1.0 / qn
    r_ = qr * qi; x_ = qx * qi; y_ = qy * qi; z_ = qz * qi
    R00 = 1 - 2 * (y_ * y_ + z_ * z_); R01 = 2 * (x_ * y_ - r_ * z_); R02 = 2 * (x_ * z_ + r_ * y_)
    R10 = 2 * (x_ * y_ + r_ * z_); R11 = 1 - 2 * (x_ * x_ + z_ * z_); R12 = 2 * (y_ * z_ - r_ * x_)
    R20 = 2 * (x_ * z_ - r_ * y_); R21 = 2 * (y_ * z_ + r_ * x_); R22 = 1 - 2 * (x_ * x_ + y_ * y_)
    s0 = sx * smod; s1 = sy * smod; s2 = sz * smod
    w0 = s0 * s0; w1 = s1 * s1; w2 = s2 * s2
    S00 = w0 * R00 * R00 + w1 * R01 * R01 + w2 * R02 * R02
    S01 = w0 * R00 * R10 + w1 * R01 * R11 + w2 * R02 * R12
    S02 = w0 * R00 * R20 + w1 * R01 * R21 + w2 * R02 * R22
    S11 = w0 * R10 * R10 + w1 * R11 * R11 + w2 * R12 * R12
    S12 = w0 * R10 * R20 + w1 * R11 * R21 + w2 * R12 * R22
    S22 = w0 * R20 * R20 + w1 * R21 * R21 + w2 * R22 * R22
    itz = 1.0 / pv2
    limx = 1.3 * tanfovx
    limy = 1.3 * tanfovy
    txc = jnp.clip(pv0 * itz, -limx, limx) * pv2
    tyc = jnp.clip(pv1 * itz, -limy, limy) * pv2
    j00 = focal_x * itz; j02 = -focal_x * txc * (itz * itz)
    j11 = focal_y * itz; j12 = -focal_y * tyc * (itz * itz)
    T00 = j00 * v[0][0] + j02 * v[0][2]
    T01 = j00 * v[1][0] + j02 * v[1][2]
    T02 = j00 * v[2][0] + j02 * v[2][2]
    T10 = j11 * v[0][1] + j12 * v[0][2]
    T11 = j11 * v[1][1] + j12 * v[1][2]
    T12 = j11 * v[2][1] + j12 * v[2][2]
    aa = (T00 * T00 * S00 + T01 * T01 * S11 + T02 * T02 * S22
          + 2.0 * (T00 * T01 * S01 + T00 * T02 * S02 + T01 * T02 * S12))
    bb = (T00 * T10 * S00 + T01 * T11 * S11 + T02 * T12 * S22
          + (T00 * T11 + T01 * T10) * S01 + (T00 * T12 + T02 * T10) * S02
          + (T01 * T12 + T02 * T11) * S12)
    cc = (T10 * T10 * S00 + T11 * T11 * S11 + T12 * T12 * S22
          + 2.0 * (T10 * T11 * S01 + T10 * T12 * S02 + T11 * T12 * S12))
    a = aa + 0.3
    b = bb
    c = cc + 0.3
    det = a * c - b * b
    det_safe = jnp.where(jnp.abs(det) < 1e-12, 1.0, det)
    invd = 1.0 / det_safe
    c0 = c * invd; c1 = -b * invd; c2 = a * invd
    mid = 0.5 * (a + c)
    disc = jnp.sqrt(jnp.maximum(0.1, mid * mid - det))
    lam1 = mid + disc
    radius = jnp.ceil(3.0 * jnp.sqrt(jnp.maximum(lam1, 1e-8)))
    pxr = ((prx + 1.0) * width - 1.0) * 0.5
    pyr = ((pry + 1.0) * height - 1.0) * 0.5
    ddx = mx - cx; ddy = my - cy; ddz = mz - cz
    di = 1.0 / (jnp.sqrt(ddx * ddx + ddy * ddy + ddz * ddz) + 1e-8)
    dxn = ddx * di; dyn = ddy * di; dzn = ddz * di
    xx = dxn * dxn; yy = dyn * dyn; zz = dzn * dzn
    xy = dxn * dyn; yz = dyn * dzn; xz = dxn * dzn
    bas = [None] * 16
    bas[1] = -SH_C1 * dyn; bas[2] = SH_C1 * dzn; bas[3] = -SH_C1 * dxn
    bas[4] = SH_C2[0] * xy; bas[5] = SH_C2[1] * yz
    bas[6] = SH_C2[2] * (2.0 * zz - xx - yy)
    bas[7] = SH_C2[3] * xz; bas[8] = SH_C2[4] * (xx - yy)
    bas[9] = SH_C3[0] * dyn * (3.0 * xx - yy)
    bas[10] = SH_C3[1] * xy * dzn
    bas[11] = SH_C3[2] * dyn * (4.0 * zz - xx - yy)
    bas[12] = SH_C3[3] * dzn * (2.0 * zz - 3.0 * xx - 3.0 * yy)
    bas[13] = SH_C3[4] * dxn * (4.0 * zz - xx - yy)
    bas[14] = SH_C3[5] * dzn * (xx - yy)
    bas[15] = SH_C3[6] * dxn * (xx - 3.0 * yy)
    rgb = []
    for ch in range(3):
        col = SH_C0 * shsr[ch]
        for i in range(1, 16):
            col = col + bas[i] * shsr[3 * i + ch]
        rgb.append(jnp.maximum(col + 0.5, 0.0))
    vis = (depth > 0.2) & (det > 0.0) & (radius > 0.0)
    visf = jnp.where(vis, 1.0, 0.0)
    radf = jnp.where(vis, radius, 0.0)
    op_eff = jnp.where(vis, opa, 0.0)
    midc = 0.5 * (c0 + c2)
    detc = c0 * c2 - c1 * c1
    lminc = midc - jnp.sqrt(jnp.maximum(midc * midc - detc, 0.0))
    thresh = 2.0 * jnp.log(255.0 * jnp.maximum(op_eff, 1e-30))
    radc = jnp.sqrt(jnp.maximum(thresh, 0.0)
                    / jnp.maximum(lminc, 1e-30)) * 1.001 + 0.1
    okf = (op_eff > 0.0) & (thresh > 0.0) & (lminc > 0.0)
    badf = jnp.where(okf & jnp.logical_not(radc <= jnp.float32(_TILE)), 1.0, 0.0)
    inv_t = 1.0 / _TILE
    g = jnp.float32(_TGRID - 1)
    tx0 = jnp.clip(jnp.floor((pxr - radc) * inv_t), 0.0, g)
    tx1 = jnp.clip(jnp.floor((pxr + radc) * inv_t), 0.0, g)
    ty0 = jnp.clip(jnp.floor((pyr - radc) * inv_t), 0.0, g)
    ty1 = jnp.clip(jnp.floor((pyr + radc) * inv_t), 0.0, g)
    one = jnp.full_like(pxr, 1.0)
    for i, row in enumerate([pxr, pyr, c0, c1, c2, op_eff, rgb[0], rgb[1],
                             rgb[2], depth, one, tx0, tx1, ty0, ty1,
                             jnp.where(okf, 1.0, 0.0)]):
        attrs[i] = row
    aux[0] = radf
    aux[1] = visf
    aux[2] = badf


def _keys_body(attrs_ref, keys_ref):
    tx0 = attrs_ref[11]; tx1 = attrs_ref[12]
    ty0 = attrs_ref[13]; ty1 = attrs_ref[14]
    ok = attrs_ref[15] > 0.5
    shp = tx0.shape
    rank = (jax.lax.broadcasted_iota(jnp.int32, shp, 0) * shp[1]
            + jax.lax.broadcasted_iota(jnp.int32, shp, 1))
    sent = jnp.int32(_NTILES << 13)
    for s in range(_KSLOT):
        txf = tx0 + jnp.float32(s % 3)
        tyf = ty0 + jnp.float32(s // 3)
        valid = ok & (txf <= tx1) & (tyf <= ty1)
        tile = (tyf * _TGRID + txf).astype(jnp.int32)
        keys_ref[s] = jnp.where(valid, (tile << 13) | rank, sent)


def _composite_body(attrs_ref, rgbd_ref, out_ref):
    i = pl.program_id(0)
    npix = _NPIX
    ch = _CH
    nchunks = attrs_ref.shape[1] // ch
    pix = i * npix + jax.lax.broadcasted_iota(jnp.int32, (npix, 1), 0)
    xf = (pix // _H).astype(jnp.float32)
    yf = (pix % _H).astype(jnp.float32)

    def shift_fill1(t, sh):
        return jnp.concatenate(
            [jnp.full((t.shape[0], sh), 1.0, t.dtype), t[:, :t.shape[1] - sh]],
            axis=1)

    def body(k, carry):
        tcar, acc = carry
        a = attrs_ref[:, pl.ds(k * ch, ch)]
        px_c = a[0:1, :]; py_c = a[1:2, :]
        c0 = a[2:3, :]; c1 = a[3:4, :]; c2 = a[4:5, :]
        opc = a[5:6, :]
        dx = px_c - xf
        dy = py_c - yf
        power = (-0.5 * (c0 * dx * dx + c2 * dy * dy)) - c1 * dx * dy
        alpha = opc * jnp.exp(jnp.minimum(power, 0.0))
        alpha = jnp.minimum(alpha, 0.99)
        alpha = jnp.where((power > 0.0) | (alpha < 1.0 / 255.0), 0.0, alpha)
        t = 1.0 - alpha
        sh = 1
        while sh < ch:
            t = t * shift_fill1(t, sh)
            sh *= 2
        tprev = tcar * shift_fill1(t, 1)
        w = jnp.where(tprev < 1e-4, 0.0, alpha * tprev)
        acc = acc + jax.lax.dot(w, rgbd_ref[pl.ds(k * ch, ch), :],
                                precision=jax.lax.Precision.HIGHEST)
        tcar = tcar * t[:, ch - 1:ch]
        return tcar, acc

    tcar0 = jnp.ones((npix, 1), jnp.float32)
    acc0 = jnp.zeros((npix, 8), jnp.float32)
    _, acc = jax.lax.fori_loop(0, nchunks, body, (tcar0, acc0))
    out_ref[...] = acc


def _composite(attrs, rgbd):
    npix_total = _W * _H
    grid = (npix_total // _NPIX,)
    return pl.pallas_call(
        _composite_body,
        grid=grid,
        in_specs=[
            pl.BlockSpec(attrs.shape, lambda i: (0, 0)),
            pl.BlockSpec(rgbd.shape, lambda i: (0, 0)),
        ],
        out_specs=pl.BlockSpec((_NPIX, 8), lambda i: (i, 0)),
        out_shape=jax.ShapeDtypeStruct((npix_total, 8), jnp.float32),
    )(attrs, rgbd)


_TILE = 16
_TGRID = _W // _TILE
_NTILES = _TGRID * _TGRID
_KSLOT = 9
_TPIX = _TILE * _TILE


def _tile_composite_body(starts_ref, binned_ref, out_ref):
    t = pl.program_id(0)
    start = starts_ref[t]
    end = starts_ref[t + 1]
    rr = jax.lax.broadcasted_iota(jnp.int32, (_TPIX, 1), 0)
    xf = ((t % _TGRID) * _TILE + rr // _TILE).astype(jnp.float32)
    yf = ((t // _TGRID) * _TILE + rr % _TILE).astype(jnp.float32)
    lane = jax.lax.broadcasted_iota(jnp.int32, (1, _CH), 1)

    def shift_fill1(v, sh):
        return jnp.concatenate(
            [jnp.full((v.shape[0], sh), 1.0, v.dtype), v[:, :v.shape[1] - sh]],
            axis=1)

    def chunk(j, carry):
        tcar, acc = carry
        a = binned_ref[j]
        o = j * _CH + lane
        valid = (o >= start) & (o < end)
        px_c = a[0:1, :]; py_c = a[1:2, :]
        c0 = a[2:3, :]; c1 = a[3:4, :]; c2 = a[4:5, :]
        opc = a[5:6, :]
        dx = px_c - xf
        dy = py_c - yf
        power = (-0.5 * (c0 * dx * dx + c2 * dy * dy)) - c1 * dx * dy
        alpha = opc * jnp.exp(jnp.minimum(power, 0.0))
        alpha = jnp.minimum(alpha, 0.99)
        alpha = jnp.where((power > 0.0) | (alpha < 1.0 / 255.0) | (~valid),
                          0.0, alpha)
        tv = 1.0 - alpha
        sh = 1
        while sh < _CH:
            tv = tv * shift_fill1(tv, sh)
            sh *= 2
        tprev = tcar * shift_fill1(tv, 1)
        w = jnp.where(tprev < 1e-4, 0.0, alpha * tprev)
        acc = acc + jax.lax.dot_general(
            w, a[6:14, :], (((1,), (1,)), ((), ())),
            precision=jax.lax.Precision.HIGHEST)
        tcar = tcar * tv[:, _CH - 1:_CH]
        return tcar, acc

    j0 = start // _CH
    j1 = (end + _CH - 1) // _CH
    tcar0 = jnp.ones((_TPIX, 1), jnp.float32)
    acc0 = jnp.zeros((_TPIX, 8), jnp.float32)
    _, acc = jax.lax.fori_loop(j0, j1, chunk, (tcar0, acc0))
    out_ref[0] = acc


def _tile_composite(starts, binned):
    grid_spec = pltpu.PrefetchScalarGridSpec(
        num_scalar_prefetch=1,
        grid=(_NTILES,),
        in_specs=[pl.BlockSpec(binned.shape, lambda t, s: (0, 0, 0))],
        out_specs=pl.BlockSpec((1, _TPIX, 8), lambda t, s: (t, 0, 0)),
    )
    return pl.pallas_call(
        _tile_composite_body,
        grid_spec=grid_spec,
        out_shape=jax.ShapeDtypeStruct((_NTILES, _TPIX, 8), jnp.float32),
    )(starts, binned)


def kernel(P, D, M, background, width, height, means3D, shs, opacities, scales,
           scale_modifier, rotations, viewmatrix, projmatrix, cam_pos,
           tanfovx, tanfovy):
    Pn = means3D.shape[0]
    sub = Pn // 1024
    vm = viewmatrix.astype(jnp.float32)
    pm = projmatrix.astype(jnp.float32)
    g10 = jnp.concatenate([means3D.T, scales.T, rotations.T],
                          axis=0).reshape(10, sub, 1024)
    shsr = shs.reshape(Pn, 48).T.reshape(48, sub, 1024)
    opar = opacities.reshape(Pn).reshape(1, sub, 1024)
    par = jnp.stack([jnp.float32(tanfovx), jnp.float32(tanfovy),
                     jnp.float32(scale_modifier), jnp.float32(width),
                     jnp.float32(height), cam_pos[0], cam_pos[1], cam_pos[2]])
    attrs_u, aux = pl.pallas_call(
        _prep_body,
        in_specs=[
            pl.BlockSpec(memory_space=pltpu.VMEM),
            pl.BlockSpec(memory_space=pltpu.VMEM),
            pl.BlockSpec(memory_space=pltpu.VMEM),
            pl.BlockSpec(memory_space=pltpu.SMEM),
            pl.BlockSpec(memory_space=pltpu.SMEM),
            pl.BlockSpec(memory_space=pltpu.SMEM),
        ],
        out_shape=[jax.ShapeDtypeStruct((16, sub, 1024), jnp.float32),
                   jax.ShapeDtypeStruct((3, sub, 1024), jnp.float32)],
    )(g10, shsr, opar, vm, pm, par)
    radii = aux[0].reshape(Pn)
    visible = aux[1].reshape(Pn) > 0.5
    safe = jnp.logical_not(jnp.any(aux[2] > 0.0))

    ones_col = jnp.ones((Pn, 1), dtype=jnp.float32)
    depths_sort = (jnp.concatenate([means3D, ones_col], axis=1) @ vm)[:, 2]
    order = jnp.argsort(depths_sort)
    attrs16 = attrs_u.reshape(16, Pn)[:, order]

    keys = pl.pallas_call(
        _keys_body,
        out_shape=jax.ShapeDtypeStruct((_KSLOT, sub, 1024), jnp.int32),
    )(attrs16.reshape(16, sub, 1024))
    keys = jnp.sort(keys.ravel())
    tile_arr = keys >> 13
    starts = jnp.searchsorted(
        tile_arr, jnp.arange(_NTILES + 1, dtype=jnp.int32)).astype(jnp.int32)
    idx = keys & (Pn - 1)
    ncap = (_KSLOT * Pn) // _CH
    binned = attrs16[:, idx].reshape(16, ncap, _CH).swapaxes(0, 1)

    def tiled_path():
        acc = _tile_composite(starts, binned)
        a = acc.reshape(_TGRID, _TGRID, _TILE, _TILE, 8)
        return a.transpose(1, 2, 0, 3, 4).reshape(_W * _H, 8)

    def dense_path():
        rgbd = attrs16[6:11].T
        rgbd = jnp.concatenate([rgbd, jnp.zeros((Pn, 3), jnp.float32)], axis=1)
        return _composite(attrs16[0:8], rgbd)

    acc = jax.lax.cond(safe, tiled_path, dense_path)
    accw = acc[:, 4:5]
    out_color = (acc[:, 0:3] + (1.0 - accw) * background[None, :]).reshape(_W, _H, 3)
    out_depth = acc[:, 3:4].reshape(_W, _H, 1)
    return out_color, out_depth, radii, visible
